# baseline (speedup 1.0000x reference)
import functools

import jax
import jax.numpy as jnp
from jax import lax
from jax.experimental import pallas as pl
from jax.experimental.pallas import tpu as pltpu
from jax.experimental.pallas import tpu_sc as plsc

N_NODES = 10000
N_PAD = 10240
N_EDGES = 160000
E_PAD = 163840
BE = 4096
BN = 1024
NEG_INF = float("-inf")

SC_CORES = 2
SC_SUBCORES = 16
SC_WORKERS = SC_CORES * SC_SUBCORES
EPW = E_PAD // SC_WORKERS
GCH = 512


def _sc_mesh():
    return plsc.VectorSubcoreMesh(
        core_axis_name="c", subcore_axis_name="s",
        num_cores=SC_CORES, num_subcores=SC_SUBCORES)


def _gather_body(q_hbm, dst_hbm, src_hbm, qd_hbm, qs_hbm,
                 di_v, si_v, rows_v, sem):
    wid = lax.axis_index("s") * SC_CORES + lax.axis_index("c")
    base = wid * EPW

    def chunk(j, _):
        off = base + j * GCH
        pltpu.sync_copy(dst_hbm.at[pl.ds(off, GCH)], di_v)
        pltpu.async_copy(q_hbm.at[di_v], rows_v, sem).wait()
        pltpu.sync_copy(rows_v, qd_hbm.at[pl.ds(off, GCH)])
        pltpu.sync_copy(src_hbm.at[pl.ds(off, GCH)], si_v)
        pltpu.async_copy(q_hbm.at[si_v], rows_v, sem).wait()
        pltpu.sync_copy(rows_v, qs_hbm.at[pl.ds(off, GCH)])
        return ()

    lax.fori_loop(0, EPW // GCH, chunk, (), unroll=False)


def _sc_gather(q, dst, src):
    run = pl.kernel(
        _gather_body,
        out_type=[
            jax.ShapeDtypeStruct((E_PAD, 128), jnp.float32),
            jax.ShapeDtypeStruct((E_PAD, 128), jnp.float32),
        ],
        mesh=_sc_mesh(),
        scratch_types=[
            pltpu.VMEM((GCH,), jnp.int32),
            pltpu.VMEM((GCH,), jnp.int32),
            pltpu.VMEM((GCH, 128), jnp.float32),
            pltpu.SemaphoreType.DMA,
        ],
    )
    return run(q, dst, src)


SCH = 8192
SROW = SCH // 128
PROW = N_PAD // 128


def _scatter_body(m3t_hbm, dst_hbm, aggt_hbm, di_v, v0_v, v1_v, p0_v, p1_v,
                  st_v):
    wid = lax.axis_index("s") * SC_CORES + lax.axis_index("c")
    c0 = wid * 2

    def init(g, _):
        p0_v[pl.ds(g * 16, 16)] = jnp.full((16,), NEG_INF, jnp.float32)
        p1_v[pl.ds(g * 16, 16)] = jnp.full((16,), NEG_INF, jnp.float32)
        return ()

    lax.fori_loop(0, N_PAD // 16, init, (), unroll=False)

    def chunk(j, _):
        off = j * SCH
        pltpu.sync_copy(dst_hbm.at[pl.ds(off, SCH)], di_v)
        pltpu.sync_copy(m3t_hbm.at[c0, pl.ds(j * SROW, SROW)], v0_v)
        pltpu.sync_copy(m3t_hbm.at[c0 + 1, pl.ds(j * SROW, SROW)], v1_v)

        def group(g, _):
            r, c8 = g // 8, g % 8
            idx = di_v[pl.ds(g * 16, 16)]
            for part, vv in ((p0_v, v0_v), (p1_v, v1_v)):
                vals = vv[r, pl.ds(c8 * 16, 16)]

                def step(mask):
                    cur = plsc.load_gather(part, [idx])
                    plsc.store_scatter(part, [idx],
                                       jnp.maximum(cur, vals), mask=mask)
                    chk = plsc.load_gather(part, [idx])
                    return chk < vals

                mask = step(jnp.ones((16,), jnp.bool_))
                lax.while_loop(jnp.any, step, mask)
            return ()

        lax.fori_loop(0, SROW * 8, group, (), unroll=False)
        return ()

    lax.fori_loop(0, E_PAD // SCH, chunk, (), unroll=False)

    for part, row in ((p0_v, c0), (p1_v, c0 + 1)):
        def cp(g, _):
            st_v[g // 8, pl.ds((g % 8) * 16, 16)] = part[pl.ds(g * 16, 16)]
            return ()
        lax.fori_loop(0, PROW * 8, cp, (), unroll=False)
        pltpu.sync_copy(st_v, aggt_hbm.at[row])


def _sc_scatter_max(m3t, dst):
    run = pl.kernel(
        _scatter_body,
        out_type=jax.ShapeDtypeStruct((64, PROW, 128), jnp.float32),
        mesh=_sc_mesh(),
        scratch_types=[
            pltpu.VMEM((SCH,), jnp.int32),
            pltpu.VMEM((SROW, 128), jnp.float32),
            pltpu.VMEM((SROW, 128), jnp.float32),
            pltpu.VMEM((N_PAD,), jnp.float32),
            pltpu.VMEM((N_PAD,), jnp.float32),
            pltpu.VMEM((PROW, 128), jnp.float32),
        ],
        compiler_params=pltpu.CompilerParams(needs_layout_passes=False),
    )
    return run(m3t, dst)


def _full(shape):
    return pl.BlockSpec(shape, lambda i: tuple(0 for _ in shape))


def _rows(block, ncols):
    return pl.BlockSpec((block, ncols), lambda i: (i, 0))


def _node_proj_body(x_ref, w_ref, q_ref):
    q_ref[...] = jnp.dot(x_ref[...], w_ref[...],
                         preferred_element_type=jnp.float32)


def _node_proj(x, wcat):
    n, k = x.shape
    grid = n // BN
    return pl.pallas_call(
        _node_proj_body,
        grid=(grid,),
        in_specs=[_rows(BN, k), _full((k, 128))],
        out_specs=_rows(BN, 128),
        out_shape=jax.ShapeDtypeStruct((n, 128), jnp.float32),
    )(x, wcat)


def _edge_mlp_body(qd_ref, qs_ref, ea_ref, we_ref, b1_ref, w2_ref, b2_ref,
                   w3_ref, b3_ref, out_ref):
    u = (qd_ref[:, :64] + qs_ref[:, 64:]
         + jnp.dot(ea_ref[...], we_ref[...], preferred_element_type=jnp.float32)
         + b1_ref[...])
    u = jnp.maximum(u, 0.0)
    u = jnp.dot(u, w2_ref[...], preferred_element_type=jnp.float32) + b2_ref[...]
    u = jnp.maximum(u, 0.0)
    out_ref[...] = (lax.dot_general(
        w3_ref[...], u, (((0,), (1,)), ((), ())),
        preferred_element_type=jnp.float32) + b3_ref[...])


def _edge_mlp(qd, qs, ea, we, b1, w2, b2, w3, b3):
    grid = E_PAD // BE
    return pl.pallas_call(
        _edge_mlp_body,
        grid=(grid,),
        in_specs=[
            _rows(BE, 128), _rows(BE, 128), _rows(BE, 16),
            _full((16, 64)), _full((1, 64)),
            _full((64, 64)), _full((1, 64)),
            _full((64, 64)), _full((64, 1)),
        ],
        out_specs=pl.BlockSpec((64, BE), lambda i: (0, i)),
        out_shape=jax.ShapeDtypeStruct((64, E_PAD), jnp.float32),
    )(qd, qs, ea, we, b1.reshape(1, 64), w2, b2.reshape(1, 64), w3,
      b3.reshape(64, 1))


def _gamma_body(aggt_ref, x_ref, wa_ref, wx_ref, b1_ref, w2_ref, b2_ref,
                w3_ref, b3_ref, out_ref, *, relu_out):
    aggt = aggt_ref[...]
    aggt = jnp.where(aggt == NEG_INF, 0.0, aggt)
    u = (lax.dot_general(aggt, wa_ref[...], (((0,), (0,)), ((), ())),
                         preferred_element_type=jnp.float32)
         + jnp.dot(x_ref[...], wx_ref[...], preferred_element_type=jnp.float32)
         + b1_ref[...])
    u = jnp.maximum(u, 0.0)
    u = jnp.dot(u, w2_ref[...], preferred_element_type=jnp.float32) + b2_ref[...]
    u = jnp.maximum(u, 0.0)
    u = jnp.dot(u, w3_ref[...], preferred_element_type=jnp.float32) + b3_ref[...]
    if relu_out:
        u = jnp.maximum(u, 0.0)
    out_ref[...] = u


def _gamma(aggt, x, wa, wx, b1, w2, b2, w3, b3, relu_out):
    n, k = x.shape
    grid = n // BN
    body = functools.partial(_gamma_body, relu_out=relu_out)
    return pl.pallas_call(
        body,
        grid=(grid,),
        in_specs=[
            pl.BlockSpec((64, BN), lambda i: (0, i)), _rows(BN, k),
            _full((64, 64)), _full((k, 64)), _full((1, 64)),
            _full((64, 64)), _full((1, 64)),
            _full((64, 64)), _full((1, 64)),
        ],
        out_specs=_rows(BN, 64),
        out_shape=jax.ShapeDtypeStruct((n, 64), jnp.float32),
    )(aggt, x, wa, wx, b1.reshape(1, 64), w2, b2.reshape(1, 64), w3,
      b3.reshape(1, 64))


def _head_body(h_ref, w1_ref, b1_ref, w2_ref, b2_ref, w3_ref, b3_ref, out_ref):
    u = jnp.dot(h_ref[...], w1_ref[...], preferred_element_type=jnp.float32) + b1_ref[...]
    u = jnp.maximum(u, 0.0)
    u = jnp.dot(u, w2_ref[...], preferred_element_type=jnp.float32) + b2_ref[...]
    u = jnp.maximum(u, 0.0)
    out_ref[...] = jnp.dot(u, w3_ref[...], preferred_element_type=jnp.float32) + b3_ref[...]


def _head(h, params):
    (w1, b1), (w2, b2), (w3, b3) = params
    grid = h.shape[0] // BN
    return pl.pallas_call(
        _head_body,
        grid=(grid,),
        in_specs=[
            _rows(BN, 64),
            _full((64, 64)), _full((1, 64)),
            _full((64, 64)), _full((1, 64)),
            _full((64, 16)), _full((1, 16)),
        ],
        out_specs=_rows(BN, 16),
        out_shape=jax.ShapeDtypeStruct((h.shape[0], 16), jnp.float32),
    )(h, w1, b1.reshape(1, 64), w2, b2.reshape(1, 64), w3, b3.reshape(1, 16))


def _layer(x, edge_attr, src, dst, phi, gamma, relu_out):
    (wp1, bp1), (wp2, bp2), (wp3, bp3) = phi
    (wg1, bg1), (wg2, bg2), (wg3, bg3) = gamma
    k = x.shape[1]
    wi = wp1[:k]
    wj = wp1[k:2 * k]
    we = wp1[2 * k:]
    q = _node_proj(x, jnp.concatenate([wi, wj], axis=1))

    qd, qs = _sc_gather(q, dst, src)
    m3t = _edge_mlp(qd, qs, edge_attr, we, bp1, wp2, bp2, wp3, bp3)

    aggt = _sc_scatter_max(m3t.reshape(64, E_PAD // 128, 128),
                           dst).reshape(64, N_PAD)

    wa = wg1[:64]
    wx = wg1[64:]
    return _gamma(aggt, x, wa, wx, bg1, wg2, bg2, wg3, bg3, relu_out)


def kernel(x, edge_attr, edge_index, params):
    phi1, gamma1, phi2, gamma2, head = params
    npad = E_PAD - N_EDGES
    src = jnp.pad(edge_index[0].astype(jnp.int32), (0, npad))
    dst = jnp.pad(edge_index[1].astype(jnp.int32), (0, npad),
                  constant_values=N_NODES)
    edge_attr = jnp.pad(edge_attr, ((0, npad), (0, 0)))
    x = jnp.pad(x, ((0, N_PAD - N_NODES), (0, 0)))
    h = _layer(x, edge_attr, src, dst, phi1, gamma1, relu_out=True)
    h = _layer(h, edge_attr, src, dst, phi2, gamma2, relu_out=False)
    return _head(h, head)[:N_NODES]
